# baseline (speedup 1.0000x reference)
import functools

import jax
import jax.numpy as jnp
from jax import lax
from jax.experimental import pallas as pl
from jax.experimental.pallas import tpu as pltpu
from jax.experimental.pallas import tpu_sc as plsc

_B, _L, _V = 32, 32, 32001
_MASK_IDX = 32000
_N = _B * _L
_R = 32
_NB = _N // _R

_NW = 32
_RW = _N // _NW
_LANES = 16


def _tc_body(src_ref, flag_ref, xt_ref, x_ref, o_ref):
    i = pl.program_id(0)
    neg_inf = jnp.float32(-jnp.inf)

    @pl.when(flag_ref[i] == 0)
    def _no_masked_rows():
        o_ref[...] = jnp.full((_R, _V), neg_inf, jnp.float32)

    @pl.when(flag_ref[i] != 0)
    def _has_masked_rows():
        x = x_ref[...]
        xt = xt_ref[...]
        lane = jax.lax.broadcasted_iota(jnp.int32, (_R, _V), 1)
        valid = lane < _MASK_IDX
        xm = jnp.where(valid, x, neg_inf)
        m = jnp.max(xm, axis=-1, keepdims=True)
        s = jnp.sum(jnp.exp(xm - m), axis=-1, keepdims=True)
        lse = m + jnp.log(s)
        sm = jnp.where(valid, x - lse, neg_inf)
        o_ref[...] = jnp.where(xt != _MASK_IDX, neg_inf, sm)


def _tc_fill(x, xt2):
    blk_has = jnp.any((xt2[:, 0] == _MASK_IDX).reshape(_NB, _R), axis=1)
    flags = blk_has.astype(jnp.int32)
    src = jax.lax.cummax(
        jnp.where(blk_has, jnp.arange(_NB, dtype=jnp.int32), 0))
    grid_spec = pltpu.PrefetchScalarGridSpec(
        num_scalar_prefetch=2,
        grid=(_NB,),
        in_specs=[
            pl.BlockSpec((_R, 1), lambda i, src_ref, flag_ref: (i, 0)),
            pl.BlockSpec((_R, _V),
                         lambda i, src_ref, flag_ref: (src_ref[i], 0)),
        ],
        out_specs=pl.BlockSpec((_R, _V), lambda i, src_ref, flag_ref: (i, 0)),
    )
    return pl.pallas_call(
        _tc_body,
        grid_spec=grid_spec,
        out_shape=jax.ShapeDtypeStruct((_N, _V), jnp.float32),
    )(src, flags, xt2, x)


@functools.partial(
    pl.kernel,
    mesh=plsc.VectorSubcoreMesh(core_axis_name="c", subcore_axis_name="s"),
    out_type=(),
    scratch_types=[
        pltpu.VMEM((_RW,), jnp.int32),
        pltpu.VMEM((_RW,), jnp.int32),
        pltpu.VMEM((_RW,), jnp.float32),
        pltpu.SemaphoreType.DMA,
    ],
)
def _sc_scatter(xt_hbm, buf_ref, xt_v, idx_v, val_v, sem):
    wid = lax.axis_index("s") * 2 + lax.axis_index("c")
    base = wid * _RW
    pltpu.sync_copy(xt_hbm.at[pl.ds(base, _RW)], xt_v)
    for j in range(_RW // _LANES):
        xv = xt_v[pl.ds(j * _LANES, _LANES)]
        rid = base + j * _LANES + lax.iota(jnp.int32, _LANES)
        idx_v[pl.ds(j * _LANES, _LANES)] = rid * _V + xv
        val_v[pl.ds(j * _LANES, _LANES)] = jnp.where(
            xv == _MASK_IDX, jnp.float32(-jnp.inf), jnp.float32(0.0))
    pltpu.async_copy(val_v, buf_ref.at[idx_v], sem).wait()


def kernel(output, xt):
    x = output.reshape(_N, _V)
    xt2 = xt.reshape(_N, 1)
    filled = _tc_fill(x, xt2)
    buf = jax.new_ref(filled.reshape(_N * _V))
    _sc_scatter(xt2.reshape(_N), buf)
    return jax.freeze(buf).reshape(_B, _L, _V)


# speedup vs baseline: 26.9667x; 26.9667x over previous
import functools

import jax
import jax.numpy as jnp
from jax import lax
from jax.experimental import pallas as pl
from jax.experimental.pallas import tpu as pltpu
from jax.experimental.pallas import tpu_sc as plsc

_B, _L, _V = 32, 32, 32001
_MASK_IDX = 32000
_N = _B * _L
_R = 32
_NB = _N // _R
_LANES = 16


@functools.partial(
    pl.kernel,
    mesh=plsc.VectorSubcoreMesh(core_axis_name="c", subcore_axis_name="s"),
    out_type=(jax.ShapeDtypeStruct((_NB,), jnp.int32),
              jax.ShapeDtypeStruct((_NB,), jnp.int32)),
    scratch_types=[
        pltpu.VMEM((_N,), jnp.int32),
        pltpu.VMEM((_NB,), jnp.int32),
        pltpu.VMEM((_NB,), jnp.int32),
    ],
    compiler_params=pltpu.CompilerParams(needs_layout_passes=False),
)
def _sc_route(xt_hbm, src_hbm, flag_hbm, xt_v, src_v, flag_v):
    wid = lax.axis_index("s") * 2 + lax.axis_index("c")

    @pl.when(wid == 0)
    def _():
        pltpu.sync_copy(xt_hbm, xt_v)
        ids = lax.iota(jnp.int32, _LANES)
        zeros = jnp.zeros((_LANES,), jnp.int32)
        f0 = zeros
        f1 = zeros
        s0 = zeros
        s1 = zeros
        lastv = zeros
        for b in range(_NB):
            m0 = xt_v[pl.ds(b * _R, _LANES)] == _MASK_IDX
            m1 = xt_v[pl.ds(b * _R + _LANES, _LANES)] == _MASK_IDX
            nhit = plsc.all_reduce_population_count(m0 | m1)
            hs = jnp.where(nhit > 0, jnp.full((_LANES,), 1, jnp.int32),
                           zeros)
            lastv = jnp.maximum(lastv, hs * b)
            if b < _LANES:
                f0 = f0 | jnp.where(ids == b, hs, zeros)
                s0 = s0 | jnp.where(ids == b, lastv, zeros)
            else:
                f1 = f1 | jnp.where(ids == b - _LANES, hs, zeros)
                s1 = s1 | jnp.where(ids == b - _LANES, lastv, zeros)
        flag_v[pl.ds(0, _LANES)] = f0
        flag_v[pl.ds(_LANES, _LANES)] = f1
        src_v[pl.ds(0, _LANES)] = s0
        src_v[pl.ds(_LANES, _LANES)] = s1
        pltpu.sync_copy(src_v, src_hbm)
        pltpu.sync_copy(flag_v, flag_hbm)


def _tc_body(src_ref, flag_ref, xt_ref, x_ref, o_ref):
    i = pl.program_id(0)
    neg_inf = jnp.float32(-jnp.inf)
    xt = xt_ref[...]
    lane = jax.lax.broadcasted_iota(jnp.int32, (_R, _V), 1)
    onehot = jnp.where(lane == xt, jnp.float32(0.0), neg_inf)

    @pl.when(flag_ref[i] == 0)
    def _no_masked_rows():
        o_ref[...] = onehot

    @pl.when(flag_ref[i] != 0)
    def _has_masked_rows():
        x = x_ref[...]
        valid = lane < _MASK_IDX
        xm = jnp.where(valid, x, neg_inf)
        m = jnp.max(xm, axis=-1, keepdims=True)
        s = jnp.sum(jnp.exp(xm - m), axis=-1, keepdims=True)
        lse = m + jnp.log(s)
        sm = jnp.where(valid, x - lse, neg_inf)
        o_ref[...] = jnp.where(xt != _MASK_IDX, onehot, sm)


def kernel(output, xt):
    x = output.reshape(_N, _V)
    xt2 = xt.reshape(_N, 1)
    src, flags = _sc_route(xt.reshape(_N))
    grid_spec = pltpu.PrefetchScalarGridSpec(
        num_scalar_prefetch=2,
        grid=(_NB,),
        in_specs=[
            pl.BlockSpec((_R, 1), lambda i, src_ref, flag_ref: (i, 0)),
            pl.BlockSpec((_R, _V),
                         lambda i, src_ref, flag_ref: (src_ref[i], 0)),
        ],
        out_specs=pl.BlockSpec((_R, _V), lambda i, src_ref, flag_ref: (i, 0)),
    )
    out = pl.pallas_call(
        _tc_body,
        grid_spec=grid_spec,
        out_shape=jax.ShapeDtypeStruct((_N, _V), jnp.float32),
    )(src, flags, xt2, x)
    return out.reshape(_B, _L, _V)
